# baseline (speedup 1.0000x reference)
import jax
import jax.numpy as jnp
from jax import lax
from jax.experimental import pallas as pl
from jax.experimental.pallas import tpu as pltpu
from jax.experimental.pallas import tpu_sc as plsc

NC, NS = 2, 16
NW = NC * NS
LANES = 128
NB = 8
NBA = 4
D = 18
DP = 24


def _fill_f32(ref, nrow16, value):
    for k in range(nrow16):
        ref[pl.ds(k * 16, 16)] = jnp.full((16,), value, jnp.float32)


def _deg_body(src2d, dst2d, od_out, id_out, *rest):
    sidx, didx, ones2, zv, odeg, ideg, sem_i, sem_a = rest
    c = lax.axis_index("c")
    s = lax.axis_index("s")
    w = s * NC + c
    nrows = src2d.shape[0]
    nsc = nrows // NB
    pad = odeg.shape[0] // NS
    _fill_f32(ones2, LANES // 16, 1.0)
    _fill_f32(zv, LANES // 16, 0.0)

    def zbody(k, carry):
        pltpu.sync_copy(zv, odeg.at[pl.ds(s * pad + k * LANES, LANES)])
        pltpu.sync_copy(zv, ideg.at[pl.ds(s * pad + k * LANES, LANES)])
        return carry

    lax.fori_loop(0, pad // LANES, zbody, 0)
    rem = pad % LANES
    if rem:
        pltpu.sync_copy(
            zv.at[pl.ds(0, rem)], odeg.at[pl.ds(s * pad + pad - rem, rem)]
        )
        pltpu.sync_copy(
            zv.at[pl.ds(0, rem)], ideg.at[pl.ds(s * pad + pad - rem, rem)]
        )
    plsc.subcore_barrier()
    iters = (nsc + NW - 1) // NW

    def body(i, carry):
        scid = w + NW * i

        @pl.when(scid < nsc)
        def _():
            base = scid * NB
            lds = [
                pltpu.async_copy(src2d.at[pl.ds(base, NB)], sidx, sem_i),
                pltpu.async_copy(dst2d.at[pl.ds(base, NB)], didx, sem_i),
            ]
            for d_ in lds:
                d_.wait()
            def fb(b, carry):
                pltpu.async_copy(ones2, odeg.at[sidx.at[b]], sem_a, add=True)
                pltpu.async_copy(ones2, ideg.at[didx.at[b]], sem_a, add=True)
                return carry

            lax.fori_loop(0, NB, fb, 0)

            def db(b, carry):
                pltpu.make_async_copy(ones2, odeg.at[sidx.at[b]], sem_a).wait()
                pltpu.make_async_copy(ones2, ideg.at[didx.at[b]], sem_a).wait()
                return carry

            lax.fori_loop(0, NB, db, 0)

        return carry

    lax.fori_loop(0, iters, body, 0)
    plsc.subcore_barrier()
    pltpu.sync_copy(odeg.at[pl.ds(s * pad, pad)], od_out.at[c, pl.ds(s * pad, pad)])
    pltpu.sync_copy(ideg.at[pl.ds(s * pad, pad)], id_out.at[c, pl.ds(s * pad, pad)])


def _agg_body(src2d, dstab, hw, zsrc, agg_out, *rest):
    sidx = rest[0:3]
    didx = rest[3:6]
    rows = rest[6:9]
    zrow = rest[9]
    agg = rest[10]
    sem_i = rest[11:14]
    sem_g = rest[14:17]
    sem_a = rest[17:20]
    c = lax.axis_index("c")
    s = lax.axis_index("s")
    nrows = src2d.shape[0]
    nsc = nrows // NBA
    npt = agg.shape[0] // NS
    pltpu.sync_copy(zsrc, zrow)

    def zbody(k, carry):
        pltpu.sync_copy(zrow, agg.at[pl.ds(s * npt + k * LANES, LANES)])
        return carry

    lax.fori_loop(0, npt // LANES, zbody, 0)
    plsc.subcore_barrier()
    dbase = c * nrows

    def valid(j):
        return s + NS * j < nsc

    def fire_idx(j, st):
        base = (s + NS * j) * NBA
        pltpu.async_copy(src2d.at[pl.ds(base, NBA)], sidx[st], sem_i[st])
        pltpu.async_copy(dstab.at[pl.ds(dbase + base, NBA)], didx[st], sem_i[st])

    def drain_idx(st):
        pltpu.make_async_copy(src2d.at[pl.ds(0, NBA)], sidx[st], sem_i[st]).wait()
        pltpu.make_async_copy(dstab.at[pl.ds(0, NBA)], didx[st], sem_i[st]).wait()

    def fire_gather(st):
        def fb(b, carry):
            pltpu.async_copy(hw.at[sidx[st].at[b]], rows[st].at[b], sem_g[st])
            return carry

        lax.fori_loop(0, NBA, fb, 0)

    def drain_gather(st):
        def fb(b, carry):
            pltpu.make_async_copy(
                hw.at[sidx[st].at[b]], rows[st].at[b], sem_g[st]
            ).wait()
            return carry

        lax.fori_loop(0, NBA, fb, 0)

    def fire_scatter(st):
        def fb(b, carry):
            pltpu.async_copy(
                rows[st].at[b], agg.at[didx[st].at[b]], sem_a[st], add=True
            )
            return carry

        lax.fori_loop(0, NBA, fb, 0)

    def drain_scatter(st):
        def fb(b, carry):
            pltpu.make_async_copy(
                rows[st].at[b], agg.at[didx[st].at[b]], sem_a[st]
            ).wait()
            return carry

        lax.fori_loop(0, NBA, fb, 0)

    @pl.when(valid(0))
    def _():
        fire_idx(0, 0)
        drain_idx(0)
        fire_gather(0)

    @pl.when(valid(1))
    def _():
        fire_idx(1, 1)

    nphase = ((nsc + NS - 1) // NS + 2 + 2) // 3

    def body(i, carry):
        for p in range(3):
            jj = 3 * i + p

            @pl.when((jj >= 1) & valid(jj - 1))
            def _(p=p):
                drain_scatter((p + 2) % 3)

            @pl.when(valid(jj + 1))
            def _(p=p, jj=jj):
                drain_idx((p + 1) % 3)
                fire_gather((p + 1) % 3)

            @pl.when(valid(jj + 2))
            def _(p=p, jj=jj):
                fire_idx(jj + 2, (p + 2) % 3)

            @pl.when(valid(jj))
            def _(p=p):
                drain_gather(p)
                fire_scatter(p)

        return carry

    lax.fori_loop(0, nphase, body, 0)
    plsc.subcore_barrier()
    pltpu.sync_copy(agg.at[pl.ds(s * npt, npt)], agg_out.at[c, pl.ds(s * npt, npt)])


def _dense1_body(nf_ref, odp_ref, feat_ref, wc_ref, wl_ref, bl_ref, hw_ref, tra_ref):
    od = odp_ref[0, :] + odp_ref[1, :]
    nrm = lax.rsqrt(jnp.maximum(od, 1.0))
    hw = jnp.dot(nf_ref[:, :], wc_ref[:, :], preferred_element_type=jnp.float32)
    hw_ref[:, :] = hw * nrm[:, None]
    tra_ref[:, :] = (
        jnp.dot(feat_ref[:, :], wl_ref[:, :], preferred_element_type=jnp.float32)
        + bl_ref[:, :]
    )


def _remap_body(hn_ref, dst_ref, out_ref):
    hn = hn_ref[0]
    trash = hn_ref[1]
    d_ = dst_ref[:, :]
    out_ref[0] = jnp.where(d_ < hn, d_, trash)
    out_ref[1] = jnp.where(d_ >= hn, d_ - hn, trash)


def _dense2_body(agg_ref, idp_ref, tra_ref, bc_ref, out_ref):
    indeg = idp_ref[0, :] + idp_ref[1, :]
    nrm = lax.rsqrt(jnp.maximum(indeg, 1.0))
    rst = jnp.maximum(agg_ref[:, :] * nrm[:, None] + bc_ref[:, :], 0.0)
    cat = jnp.concatenate([tra_ref[:, :], rst], axis=1)
    out_ref[:, :] = cat[:, : out_ref.shape[1]]


def kernel(node_feat, feat, edge_index, W_conv, b_conv, W_lin, b_lin):
    N = node_feat.shape[0]
    E = edge_index.shape[1]
    H = W_lin.shape[1]
    src2d = edge_index[0].astype(jnp.int32).reshape(E // LANES, LANES)
    dst2d = edge_index[1].astype(jnp.int32).reshape(E // LANES, LANES)
    nrows = E // LANES

    pad_tile = ((N + NS - 1) // NS + 7) // 8 * 8
    padn = NS * pad_tile
    HN = ((N + 1) // 2 + 7) // 8 * 8
    ACC = ((HN + 1) + NS * LANES - 1) // (NS * LANES) * (NS * LANES)

    mesh = plsc.VectorSubcoreMesh(
        core_axis_name="c", subcore_axis_name="s", num_cores=NC, num_subcores=NS
    )
    sc_params = pltpu.CompilerParams(use_tc_tiling_on_sc=False)

    deg_call = pl.kernel(
        _deg_body,
        out_type=[
            jax.ShapeDtypeStruct((NC, padn), jnp.float32),
            jax.ShapeDtypeStruct((NC, padn), jnp.float32),
        ],
        mesh=mesh,
        compiler_params=sc_params,
        scratch_types=[
            pltpu.VMEM((NB, LANES), jnp.int32),
            pltpu.VMEM((NB, LANES), jnp.int32),
            pltpu.VMEM((LANES,), jnp.float32),
            pltpu.VMEM((LANES,), jnp.float32),
            pltpu.VMEM_SHARED((padn,), jnp.float32),
            pltpu.VMEM_SHARED((padn,), jnp.float32),
            pltpu.SemaphoreType.DMA,
            pltpu.SemaphoreType.DMA,
        ],
    )
    odp, idp = deg_call(src2d, dst2d)
    odp = odp[:, :N]
    idp = idp[:, :N]

    wc_pad = jnp.concatenate(
        [W_conv, jnp.zeros((D, DP - D), jnp.float32)], axis=1
    )
    R = 2048
    nb_ = (N + R - 1) // R
    hw, tra = pl.pallas_call(
        _dense1_body,
        grid=(nb_,),
        in_specs=[
            pl.BlockSpec((R, D), lambda i: (i, 0)),
            pl.BlockSpec((NC, R), lambda i: (0, i)),
            pl.BlockSpec((R, feat.shape[1]), lambda i: (i, 0)),
            pl.BlockSpec((D, DP), lambda i: (0, 0)),
            pl.BlockSpec((feat.shape[1], H), lambda i: (0, 0)),
            pl.BlockSpec((1, H), lambda i: (0, 0)),
        ],
        out_specs=[
            pl.BlockSpec((R, DP), lambda i: (i, 0)),
            pl.BlockSpec((R, H), lambda i: (i, 0)),
        ],
        out_shape=[
            jax.ShapeDtypeStruct((N, DP), jnp.float32),
            jax.ShapeDtypeStruct((N, H), jnp.float32),
        ],
    )(node_feat, odp, feat, wc_pad, W_lin, b_lin.reshape(1, H))

    RB = 1000
    nrb = (nrows + RB - 1) // RB
    dstab = pl.pallas_call(
        _remap_body,
        grid=(nrb,),
        in_specs=[
            pl.BlockSpec(memory_space=pltpu.SMEM),
            pl.BlockSpec((RB, LANES), lambda i: (i, 0)),
        ],
        out_specs=pl.BlockSpec((NC, RB, LANES), lambda i: (0, i, 0)),
        out_shape=jax.ShapeDtypeStruct((NC, nrows, LANES), jnp.int32),
    )(jnp.array([HN, HN], jnp.int32), dst2d)
    dstab = dstab.reshape(NC * nrows, LANES)

    agg_call = pl.kernel(
        _agg_body,
        out_type=jax.ShapeDtypeStruct((NC, ACC, DP), jnp.float32),
        mesh=mesh,
        compiler_params=sc_params,
        scratch_types=[pltpu.VMEM((NBA, LANES), jnp.int32) for _ in range(6)]
        + [pltpu.VMEM((NBA, LANES, DP), jnp.float32) for _ in range(3)]
        + [
            pltpu.VMEM((LANES, DP), jnp.float32),
            pltpu.VMEM_SHARED((ACC, DP), jnp.float32),
        ]
        + [pltpu.SemaphoreType.DMA for _ in range(9)],
    )
    aggp = agg_call(src2d, dstab, hw, jnp.zeros((LANES, DP), jnp.float32))
    agg = jnp.concatenate([aggp[0, :HN], aggp[1, : N - HN]], axis=0)

    bc_pad = jnp.concatenate(
        [b_conv, jnp.zeros((DP - D,), jnp.float32)]
    ).reshape(1, DP)
    out = pl.pallas_call(
        _dense2_body,
        grid=(nb_,),
        in_specs=[
            pl.BlockSpec((R, DP), lambda i: (i, 0)),
            pl.BlockSpec((NC, R), lambda i: (0, i)),
            pl.BlockSpec((R, H), lambda i: (i, 0)),
            pl.BlockSpec((1, DP), lambda i: (0, 0)),
        ],
        out_specs=pl.BlockSpec((R, H + D), lambda i: (i, 0)),
        out_shape=jax.ShapeDtypeStruct((N, H + D), jnp.float32),
    )(agg, idp, tra, bc_pad)
    return out


# speedup vs baseline: 2.3215x; 2.3215x over previous
import jax
import jax.numpy as jnp
from jax import lax
from jax.experimental import pallas as pl
from jax.experimental.pallas import tpu as pltpu
from jax.experimental.pallas import tpu_sc as plsc

NC, NS = 2, 16
NW = NC * NS
LANES = 128
NB = 8
NBA = 4
D = 18
WD = 32
DG = 16


def _fill_f32(ref, nrow16, value):
    for k in range(nrow16):
        ref[pl.ds(k * 16, 16)] = jnp.full((16,), value, jnp.float32)


def _deg_body(src2d, dst2d, od_out, id_out, *rest):
    sidx, didx, ones2, zv, odeg, ideg, sem_i, sem_a = rest
    c = lax.axis_index("c")
    s = lax.axis_index("s")
    w = s * NC + c
    nrows = src2d.shape[0]
    nsc = nrows // NB
    pad = odeg.shape[0] // NS
    _fill_f32(ones2, LANES // 16, 1.0)
    _fill_f32(zv, LANES // 16, 0.0)

    def zbody(k, carry):
        pltpu.sync_copy(zv, odeg.at[pl.ds(s * pad + k * LANES, LANES)])
        pltpu.sync_copy(zv, ideg.at[pl.ds(s * pad + k * LANES, LANES)])
        return carry

    lax.fori_loop(0, pad // LANES, zbody, 0)
    rem = pad % LANES
    if rem:
        pltpu.sync_copy(
            zv.at[pl.ds(0, rem)], odeg.at[pl.ds(s * pad + pad - rem, rem)]
        )
        pltpu.sync_copy(
            zv.at[pl.ds(0, rem)], ideg.at[pl.ds(s * pad + pad - rem, rem)]
        )
    plsc.subcore_barrier()
    iters = (nsc + NW - 1) // NW

    def body(i, carry):
        scid = w + NW * i

        @pl.when(scid < nsc)
        def _():
            base = scid * NB
            lds = [
                pltpu.async_copy(src2d.at[pl.ds(base, NB)], sidx, sem_i),
                pltpu.async_copy(dst2d.at[pl.ds(base, NB)], didx, sem_i),
            ]
            for d_ in lds:
                d_.wait()
            def fb(b, carry):
                pltpu.async_copy(ones2, odeg.at[sidx.at[b]], sem_a, add=True)
                pltpu.async_copy(ones2, ideg.at[didx.at[b]], sem_a, add=True)
                return carry

            lax.fori_loop(0, NB, fb, 0)

            def db(b, carry):
                pltpu.make_async_copy(ones2, odeg.at[sidx.at[b]], sem_a).wait()
                pltpu.make_async_copy(ones2, ideg.at[didx.at[b]], sem_a).wait()
                return carry

            lax.fori_loop(0, NB, db, 0)

        return carry

    lax.fori_loop(0, iters, body, 0)
    plsc.subcore_barrier()
    pltpu.sync_copy(odeg.at[pl.ds(s * pad, pad)], od_out.at[c, pl.ds(s * pad, pad)])
    pltpu.sync_copy(ideg.at[pl.ds(s * pad, pad)], id_out.at[c, pl.ds(s * pad, pad)])


def _agg_body(srcab, dst2d, hw2, zsrc, agg_out, *rest):
    sidx = rest[0:3]
    didx = rest[3:6]
    rows = rest[6:9]
    zrow = rest[9]
    agg = rest[10]
    sem_i = rest[11:14]
    sem_g = rest[14:17]
    sem_a = rest[17:20]
    c = lax.axis_index("c")
    s = lax.axis_index("s")
    nrows = dst2d.shape[0]
    nsc = nrows // NBA
    npt = agg.shape[0] // NS
    pltpu.sync_copy(zsrc, zrow)

    def zbody(k, carry):
        pltpu.sync_copy(zrow, agg.at[pl.ds(s * npt + k * LANES, LANES)])
        return carry

    lax.fori_loop(0, npt // LANES, zbody, 0)
    plsc.subcore_barrier()
    sbase = c * nrows

    def valid(j):
        return s + NS * j < nsc

    def fire_idx(j, st):
        base = (s + NS * j) * NBA
        pltpu.async_copy(srcab.at[pl.ds(sbase + base, NBA)], sidx[st], sem_i[st])
        pltpu.async_copy(dst2d.at[pl.ds(base, NBA)], didx[st], sem_i[st])

    def drain_idx(st):
        pltpu.make_async_copy(dst2d.at[pl.ds(0, NBA)], sidx[st], sem_i[st]).wait()
        pltpu.make_async_copy(dst2d.at[pl.ds(0, NBA)], didx[st], sem_i[st]).wait()

    def fire_gather(st):
        def fb(b, carry):
            pltpu.async_copy(hw2.at[sidx[st].at[b]], rows[st].at[b], sem_g[st])
            return carry

        lax.fori_loop(0, NBA, fb, 0)

    def drain_gather(st):
        def fb(b, carry):
            pltpu.make_async_copy(
                hw2.at[sidx[st].at[b]], rows[st].at[b], sem_g[st]
            ).wait()
            return carry

        lax.fori_loop(0, NBA, fb, 0)

    def fire_scatter(st):
        def fb(b, carry):
            pltpu.async_copy(
                rows[st].at[b], agg.at[didx[st].at[b]], sem_a[st], add=True
            )
            return carry

        lax.fori_loop(0, NBA, fb, 0)

    def drain_scatter(st):
        def fb(b, carry):
            pltpu.make_async_copy(
                rows[st].at[b], agg.at[didx[st].at[b]], sem_a[st]
            ).wait()
            return carry

        lax.fori_loop(0, NBA, fb, 0)

    @pl.when(valid(0))
    def _():
        fire_idx(0, 0)
        drain_idx(0)
        fire_gather(0)

    @pl.when(valid(1))
    def _():
        fire_idx(1, 1)

    nphase = ((nsc + NS - 1) // NS + 2 + 2) // 3

    def body(i, carry):
        for p in range(3):
            jj = 3 * i + p

            @pl.when((jj >= 1) & valid(jj - 1))
            def _(p=p):
                drain_scatter((p + 2) % 3)

            @pl.when(valid(jj + 1))
            def _(p=p, jj=jj):
                drain_idx((p + 1) % 3)
                fire_gather((p + 1) % 3)

            @pl.when(valid(jj + 2))
            def _(p=p, jj=jj):
                fire_idx(jj + 2, (p + 2) % 3)

            @pl.when(valid(jj))
            def _(p=p):
                drain_gather(p)
                fire_scatter(p)

        return carry

    lax.fori_loop(0, nphase, body, 0)
    plsc.subcore_barrier()
    pltpu.sync_copy(agg.at[pl.ds(s * npt, npt)], agg_out.at[c, pl.ds(s * npt, npt)])


def _dense1_body(
    nf_ref, odp_ref, feat_ref, wc_ref, wl_ref, bl_ref, hwlo_ref, hwhi_ref, tra_ref
):
    od = odp_ref[0, :] + odp_ref[1, :]
    nrm = lax.rsqrt(jnp.maximum(od, 1.0))
    hw = jnp.dot(nf_ref[:, :], wc_ref[:, :], preferred_element_type=jnp.float32)
    hw = hw * nrm[:, None]
    hwlo_ref[:, :] = hw[:, :DG]
    hwhi_ref[:, :] = hw[:, DG:]
    tra_ref[:, :] = (
        jnp.dot(feat_ref[:, :], wl_ref[:, :], preferred_element_type=jnp.float32)
        + bl_ref[:, :]
    )


def _remap_body(hn_ref, src_ref, out_ref):
    n = hn_ref[0]
    s_ = src_ref[:, :]
    out_ref[0] = s_
    out_ref[1] = s_ + n


def _dense2_body(agg_ref, idp_ref, tra_ref, bc_ref, out_ref):
    indeg = idp_ref[0, :] + idp_ref[1, :]
    nrm = lax.rsqrt(jnp.maximum(indeg, 1.0))
    a = jnp.concatenate([agg_ref[0], agg_ref[1][:, : D - DG]], axis=1)
    rst = jnp.maximum(a * nrm[:, None] + bc_ref[:, :], 0.0)
    cat = jnp.concatenate([tra_ref[:, :], rst], axis=1)
    out_ref[:, :] = cat[:, : out_ref.shape[1]]


def kernel(node_feat, feat, edge_index, W_conv, b_conv, W_lin, b_lin):
    N = node_feat.shape[0]
    E = edge_index.shape[1]
    H = W_lin.shape[1]
    src2d = edge_index[0].astype(jnp.int32).reshape(E // LANES, LANES)
    dst2d = edge_index[1].astype(jnp.int32).reshape(E // LANES, LANES)
    nrows = E // LANES

    pad_tile = ((N + NS - 1) // NS + 7) // 8 * 8
    padn = NS * pad_tile
    ACC = (N + NS * LANES - 1) // (NS * LANES) * (NS * LANES)

    mesh = plsc.VectorSubcoreMesh(
        core_axis_name="c", subcore_axis_name="s", num_cores=NC, num_subcores=NS
    )
    sc_params = pltpu.CompilerParams(use_tc_tiling_on_sc=False)

    deg_call = pl.kernel(
        _deg_body,
        out_type=[
            jax.ShapeDtypeStruct((NC, padn), jnp.float32),
            jax.ShapeDtypeStruct((NC, padn), jnp.float32),
        ],
        mesh=mesh,
        compiler_params=sc_params,
        scratch_types=[
            pltpu.VMEM((NB, LANES), jnp.int32),
            pltpu.VMEM((NB, LANES), jnp.int32),
            pltpu.VMEM((LANES,), jnp.float32),
            pltpu.VMEM((LANES,), jnp.float32),
            pltpu.VMEM_SHARED((padn,), jnp.float32),
            pltpu.VMEM_SHARED((padn,), jnp.float32),
            pltpu.SemaphoreType.DMA,
            pltpu.SemaphoreType.DMA,
        ],
    )
    odp, idp = deg_call(src2d, dst2d)
    odp = odp[:, :N]
    idp = idp[:, :N]

    wc_pad = jnp.concatenate(
        [W_conv, jnp.zeros((D, WD - D), jnp.float32)], axis=1
    )
    R = 2048
    nb_ = (N + R - 1) // R
    hw_lo, hw_hi, tra = pl.pallas_call(
        _dense1_body,
        grid=(nb_,),
        in_specs=[
            pl.BlockSpec((R, D), lambda i: (i, 0)),
            pl.BlockSpec((NC, R), lambda i: (0, i)),
            pl.BlockSpec((R, feat.shape[1]), lambda i: (i, 0)),
            pl.BlockSpec((D, WD), lambda i: (0, 0)),
            pl.BlockSpec((feat.shape[1], H), lambda i: (0, 0)),
            pl.BlockSpec((1, H), lambda i: (0, 0)),
        ],
        out_specs=[
            pl.BlockSpec((R, DG), lambda i: (i, 0)),
            pl.BlockSpec((R, DG), lambda i: (i, 0)),
            pl.BlockSpec((R, H), lambda i: (i, 0)),
        ],
        out_shape=[
            jax.ShapeDtypeStruct((N, DG), jnp.float32),
            jax.ShapeDtypeStruct((N, DG), jnp.float32),
            jax.ShapeDtypeStruct((N, H), jnp.float32),
        ],
    )(node_feat, odp, feat, wc_pad, W_lin, b_lin.reshape(1, H))
    hw2 = jnp.concatenate([hw_lo, hw_hi], axis=0)

    RB = 1000
    nrb = (nrows + RB - 1) // RB
    srcab = pl.pallas_call(
        _remap_body,
        grid=(nrb,),
        in_specs=[
            pl.BlockSpec(memory_space=pltpu.SMEM),
            pl.BlockSpec((RB, LANES), lambda i: (i, 0)),
        ],
        out_specs=pl.BlockSpec((NC, RB, LANES), lambda i: (0, i, 0)),
        out_shape=jax.ShapeDtypeStruct((NC, nrows, LANES), jnp.int32),
    )(jnp.array([N], jnp.int32), src2d)
    srcab = srcab.reshape(NC * nrows, LANES)

    agg_call = pl.kernel(
        _agg_body,
        out_type=jax.ShapeDtypeStruct((NC, ACC, DG), jnp.float32),
        mesh=mesh,
        compiler_params=sc_params,
        scratch_types=[pltpu.VMEM((NBA, LANES), jnp.int32) for _ in range(6)]
        + [pltpu.VMEM((NBA, LANES, DG), jnp.float32) for _ in range(3)]
        + [
            pltpu.VMEM((LANES, DG), jnp.float32),
            pltpu.VMEM_SHARED((ACC, DG), jnp.float32),
        ]
        + [pltpu.SemaphoreType.DMA for _ in range(9)],
    )
    aggp = agg_call(srcab, dst2d, hw2, jnp.zeros((LANES, DG), jnp.float32))
    aggp = aggp[:, :N]

    out = pl.pallas_call(
        _dense2_body,
        grid=(nb_,),
        in_specs=[
            pl.BlockSpec((NC, R, DG), lambda i: (0, i, 0)),
            pl.BlockSpec((NC, R), lambda i: (0, i)),
            pl.BlockSpec((R, H), lambda i: (i, 0)),
            pl.BlockSpec((1, D), lambda i: (0, 0)),
        ],
        out_specs=pl.BlockSpec((R, H + D), lambda i: (i, 0)),
        out_shape=jax.ShapeDtypeStruct((N, H + D), jnp.float32),
    )(aggp, idp, tra, b_conv.reshape(1, D))
    return out


# speedup vs baseline: 2.5355x; 1.0922x over previous
import jax
import jax.numpy as jnp
from jax import lax
from jax.experimental import pallas as pl
from jax.experimental.pallas import tpu as pltpu
from jax.experimental.pallas import tpu_sc as plsc

NC, NS = 2, 16
NW = NC * NS
LANES = 128
NB = 8
NBA = 4
D = 18
WD = 32
DG = 16


def _fill_f32(ref, nrow16, value):
    for k in range(nrow16):
        ref[pl.ds(k * 16, 16)] = jnp.full((16,), value, jnp.float32)


def _deg_body(src2d, dst2d, od_out, id_out, *rest):
    sidx, didx, ones2, zv, odeg, ideg, sem_i, sem_a = rest
    c = lax.axis_index("c")
    s = lax.axis_index("s")
    w = s * NC + c
    nrows = src2d.shape[0]
    nsc = nrows // NB
    pad = odeg.shape[0] // NS
    _fill_f32(ones2, LANES // 16, 1.0)
    _fill_f32(zv, LANES // 16, 0.0)

    def zbody(k, carry):
        pltpu.sync_copy(zv, odeg.at[pl.ds(s * pad + k * LANES, LANES)])
        pltpu.sync_copy(zv, ideg.at[pl.ds(s * pad + k * LANES, LANES)])
        return carry

    lax.fori_loop(0, pad // LANES, zbody, 0)
    rem = pad % LANES
    if rem:
        pltpu.sync_copy(
            zv.at[pl.ds(0, rem)], odeg.at[pl.ds(s * pad + pad - rem, rem)]
        )
        pltpu.sync_copy(
            zv.at[pl.ds(0, rem)], ideg.at[pl.ds(s * pad + pad - rem, rem)]
        )
    plsc.subcore_barrier()
    iters = (nsc + NW - 1) // NW

    def body(i, carry):
        scid = w + NW * i

        @pl.when(scid < nsc)
        def _():
            base = scid * NB
            lds = [
                pltpu.async_copy(src2d.at[pl.ds(base, NB)], sidx, sem_i),
                pltpu.async_copy(dst2d.at[pl.ds(base, NB)], didx, sem_i),
            ]
            for d_ in lds:
                d_.wait()
            def fb(b, carry):
                pltpu.async_copy(ones2, odeg.at[sidx.at[b]], sem_a, add=True)
                pltpu.async_copy(ones2, ideg.at[didx.at[b]], sem_a, add=True)
                return carry

            lax.fori_loop(0, NB, fb, 0)

            def db(b, carry):
                pltpu.make_async_copy(ones2, odeg.at[sidx.at[b]], sem_a).wait()
                pltpu.make_async_copy(ones2, ideg.at[didx.at[b]], sem_a).wait()
                return carry

            lax.fori_loop(0, NB, db, 0)

        return carry

    lax.fori_loop(0, iters, body, 0)
    plsc.subcore_barrier()
    pltpu.sync_copy(odeg.at[pl.ds(s * pad, pad)], od_out.at[c, pl.ds(s * pad, pad)])
    pltpu.sync_copy(ideg.at[pl.ds(s * pad, pad)], id_out.at[c, pl.ds(s * pad, pad)])


def _agg_body(srcab, dst2d, hw2, zsrc, agg_out, *rest):
    sidx = rest[0:3]
    didx = rest[3:6]
    rows = rest[6:9]
    zrow = rest[9]
    agg = rest[10]
    sem_i = rest[11:14]
    sem_g = rest[14:17]
    sem_a = rest[17:20]
    c = lax.axis_index("c")
    s = lax.axis_index("s")
    nrows = dst2d.shape[0]
    nsc = nrows // NBA
    npt = agg.shape[0] // NS
    pltpu.sync_copy(zsrc, zrow)

    def zbody(k, carry):
        pltpu.sync_copy(zrow, agg.at[pl.ds(s * npt + k * LANES, LANES)])
        return carry

    lax.fori_loop(0, npt // LANES, zbody, 0)
    plsc.subcore_barrier()
    sbase = c * nrows

    def valid(j):
        return s + NS * j < nsc

    def fire_idx(j, st):
        base = (s + NS * j) * NBA
        pltpu.async_copy(srcab.at[pl.ds(sbase + base, NBA)], sidx[st], sem_i[st])
        pltpu.async_copy(dst2d.at[pl.ds(base, NBA)], didx[st], sem_i[st])

    def drain_idx(st):
        pltpu.make_async_copy(dst2d.at[pl.ds(0, NBA)], sidx[st], sem_i[st]).wait()
        pltpu.make_async_copy(dst2d.at[pl.ds(0, NBA)], didx[st], sem_i[st]).wait()

    def fire_gather(st):
        def fb(b, carry):
            pltpu.async_copy(hw2.at[sidx[st].at[b]], rows[st].at[b], sem_g[st])
            return carry

        lax.fori_loop(0, NBA, fb, 0)

    def drain_gather(st):
        def fb(b, carry):
            pltpu.make_async_copy(
                hw2.at[sidx[st].at[b]], rows[st].at[b], sem_g[st]
            ).wait()
            return carry

        lax.fori_loop(0, NBA, fb, 0)

    def fire_scatter(st):
        def fb(b, carry):
            pltpu.async_copy(
                rows[st].at[b], agg.at[didx[st].at[b]], sem_a[st], add=True
            )
            return carry

        lax.fori_loop(0, NBA, fb, 0)

    def drain_scatter(st):
        def fb(b, carry):
            pltpu.make_async_copy(
                rows[st].at[b], agg.at[didx[st].at[b]], sem_a[st]
            ).wait()
            return carry

        lax.fori_loop(0, NBA, fb, 0)

    @pl.when(valid(0))
    def _():
        fire_idx(0, 0)
        drain_idx(0)
        fire_gather(0)

    @pl.when(valid(1))
    def _():
        fire_idx(1, 1)

    nphase = ((nsc + NS - 1) // NS + 2 + 2) // 3

    def body(i, carry):
        for p in range(3):
            jj = 3 * i + p

            @pl.when((jj >= 1) & valid(jj - 1))
            def _(p=p):
                drain_scatter((p + 2) % 3)

            @pl.when(valid(jj + 1))
            def _(p=p, jj=jj):
                drain_idx((p + 1) % 3)
                fire_gather((p + 1) % 3)

            @pl.when(valid(jj + 2))
            def _(p=p, jj=jj):
                fire_idx(jj + 2, (p + 2) % 3)

            @pl.when(valid(jj))
            def _(p=p):
                drain_gather(p)
                fire_scatter(p)

        return carry

    lax.fori_loop(0, nphase, body, 0)
    plsc.subcore_barrier()
    pltpu.sync_copy(agg.at[pl.ds(s * npt, npt)], agg_out.at[c, pl.ds(s * npt, npt)])


def _dense1_body(
    nf_ref, odp_ref, feat_ref, wc_ref, wl_ref, bl_ref, hn_ref, src_ref,
    hw2_ref, tra_ref, srcab_ref,
):
    od = odp_ref[0, :] + odp_ref[1, :]
    nrm = lax.rsqrt(jnp.maximum(od, 1.0))
    hw = jnp.dot(nf_ref[:, :], wc_ref[:, :], preferred_element_type=jnp.float32)
    hw = hw * nrm[:, None]
    hw2_ref[0] = hw[:, :DG]
    hw2_ref[1] = hw[:, DG:]
    tra_ref[:, :] = (
        jnp.dot(feat_ref[:, :], wl_ref[:, :], preferred_element_type=jnp.float32)
        + bl_ref[:, :]
    )
    s_ = src_ref[:, :]
    srcab_ref[0] = s_
    srcab_ref[1] = s_ + hn_ref[0]


def _dense2_body(agg_ref, idp_ref, tra_ref, bc_ref, out_ref):
    indeg = idp_ref[0, :] + idp_ref[1, :]
    nrm = lax.rsqrt(jnp.maximum(indeg, 1.0))
    a = jnp.concatenate([agg_ref[0], agg_ref[1][:, : D - DG]], axis=1)
    rst = jnp.maximum(a * nrm[:, None] + bc_ref[:, :], 0.0)
    cat = jnp.concatenate([tra_ref[:, :], rst], axis=1)
    out_ref[:, :] = cat[:, : out_ref.shape[1]]


def kernel(node_feat, feat, edge_index, W_conv, b_conv, W_lin, b_lin):
    N = node_feat.shape[0]
    E = edge_index.shape[1]
    H = W_lin.shape[1]
    src2d = edge_index[0].astype(jnp.int32).reshape(E // LANES, LANES)
    dst2d = edge_index[1].astype(jnp.int32).reshape(E // LANES, LANES)
    nrows = E // LANES

    pad_tile = ((N + NS - 1) // NS + 7) // 8 * 8
    padn = NS * pad_tile
    ACC = (N + NS * LANES - 1) // (NS * LANES) * (NS * LANES)

    mesh = plsc.VectorSubcoreMesh(
        core_axis_name="c", subcore_axis_name="s", num_cores=NC, num_subcores=NS
    )
    sc_params = pltpu.CompilerParams(use_tc_tiling_on_sc=False)

    deg_call = pl.kernel(
        _deg_body,
        out_type=[
            jax.ShapeDtypeStruct((NC, padn), jnp.float32),
            jax.ShapeDtypeStruct((NC, padn), jnp.float32),
        ],
        mesh=mesh,
        compiler_params=sc_params,
        scratch_types=[
            pltpu.VMEM((NB, LANES), jnp.int32),
            pltpu.VMEM((NB, LANES), jnp.int32),
            pltpu.VMEM((LANES,), jnp.float32),
            pltpu.VMEM((LANES,), jnp.float32),
            pltpu.VMEM_SHARED((padn,), jnp.float32),
            pltpu.VMEM_SHARED((padn,), jnp.float32),
            pltpu.SemaphoreType.DMA,
            pltpu.SemaphoreType.DMA,
        ],
    )
    odp, idp = deg_call(src2d, dst2d)
    odp = odp[:, :N]
    idp = idp[:, :N]

    wc_pad = jnp.concatenate(
        [W_conv, jnp.zeros((D, WD - D), jnp.float32)], axis=1
    )
    R = 2048
    nb_ = (N + R - 1) // R
    RB = 1000
    nrb = (nrows + RB - 1) // RB
    hw2, tra, srcab = pl.pallas_call(
        _dense1_body,
        grid=(nb_,),
        in_specs=[
            pl.BlockSpec((R, D), lambda i: (i, 0)),
            pl.BlockSpec((NC, R), lambda i: (0, i)),
            pl.BlockSpec((R, feat.shape[1]), lambda i: (i, 0)),
            pl.BlockSpec((D, WD), lambda i: (0, 0)),
            pl.BlockSpec((feat.shape[1], H), lambda i: (0, 0)),
            pl.BlockSpec((1, H), lambda i: (0, 0)),
            pl.BlockSpec(memory_space=pltpu.SMEM),
            pl.BlockSpec((RB, LANES), lambda i: (jnp.minimum(i, nrb - 1), 0)),
        ],
        out_specs=[
            pl.BlockSpec((2, R, DG), lambda i: (0, i, 0)),
            pl.BlockSpec((R, H), lambda i: (i, 0)),
            pl.BlockSpec(
                (NC, RB, LANES), lambda i: (0, jnp.minimum(i, nrb - 1), 0)
            ),
        ],
        out_shape=[
            jax.ShapeDtypeStruct((2, N, DG), jnp.float32),
            jax.ShapeDtypeStruct((N, H), jnp.float32),
            jax.ShapeDtypeStruct((NC, nrows, LANES), jnp.int32),
        ],
    )(
        node_feat, odp, feat, wc_pad, W_lin, b_lin.reshape(1, H),
        jnp.array([N], jnp.int32), src2d,
    )
    hw2 = hw2.reshape(2 * N, DG)
    srcab = srcab.reshape(NC * nrows, LANES)

    agg_call = pl.kernel(
        _agg_body,
        out_type=jax.ShapeDtypeStruct((NC, ACC, DG), jnp.float32),
        mesh=mesh,
        compiler_params=sc_params,
        scratch_types=[pltpu.VMEM((NBA, LANES), jnp.int32) for _ in range(6)]
        + [pltpu.VMEM((NBA, LANES, DG), jnp.float32) for _ in range(3)]
        + [
            pltpu.VMEM((LANES, DG), jnp.float32),
            pltpu.VMEM_SHARED((ACC, DG), jnp.float32),
        ]
        + [pltpu.SemaphoreType.DMA for _ in range(9)],
    )
    aggp = agg_call(srcab, dst2d, hw2, jnp.zeros((LANES, DG), jnp.float32))
    aggp = aggp[:, :N]

    out = pl.pallas_call(
        _dense2_body,
        grid=(nb_,),
        in_specs=[
            pl.BlockSpec((NC, R, DG), lambda i: (0, i, 0)),
            pl.BlockSpec((NC, R), lambda i: (0, i)),
            pl.BlockSpec((R, H), lambda i: (i, 0)),
            pl.BlockSpec((1, D), lambda i: (0, 0)),
        ],
        out_specs=pl.BlockSpec((R, H + D), lambda i: (i, 0)),
        out_shape=jax.ShapeDtypeStruct((N, H + D), jnp.float32),
    )(aggp, idp, tra, b_conv.reshape(1, D))
    return out
